# baseline (speedup 1.0000x reference)
import jax
import jax.numpy as jnp
import numpy as np
from jax.experimental import pallas as pl
from jax.experimental.pallas import tpu as pltpu

B, N, L = 1, 128, 256
D_NODE, D_EDGE_IN, D_HID, D_EHID, H = 64, 128, 64, 64, 4
HD = H * D_HID

_PREC = jax.lax.Precision.DEFAULT
_TI = 64
_TJ = 32


def _elu(x):
    return jnp.where(x > 0, x, jnp.exp(x) - 1.0)


def _lnorm(x, g, b, eps=1e-5):
    m = x.mean(-1, keepdims=True)
    v = ((x - m) ** 2).mean(-1, keepdims=True)
    return (x - m) * jax.lax.rsqrt(v + eps) * g + b


def _dot(a, b):
    return jax.lax.dot_general(a, b, (((a.ndim - 1,), (0,)), ((), ())),
                               precision=_PREC,
                               preferred_element_type=jnp.float32)


def _dg(a, b, ca, cb):
    return jax.lax.dot_general(a, b, (((ca,), (cb,)), ((), ())),
                               precision=_PREC,
                               preferred_element_type=jnp.float32)


def _node_kernel(msa_ref, seq_ref, nng_ref, nnb_ref, swq_ref, swbq_ref,
                 swk_ref, swbk_ref, exw1_ref, exw2_ref, exb_ref, out_ref):
    tl = msa_ref.shape[1]
    m = _lnorm(msa_ref[...], nng_ref[0], nnb_ref[0])
    tar = m[0]
    q = _dot(tar, swq_ref[...]) + swbq_ref[0]
    k = (_dot(m.reshape(N * tl, D_NODE), swk_ref[...])
         + swbk_ref[0]).reshape(N, tl, D_NODE)
    logits = (q[None, :, :] * k).sum(-1) * (1.0 / 8.0)
    amax = logits.max(axis=0, keepdims=True)
    ex = jnp.exp(logits - amax)
    attn = ex / ex.sum(axis=0, keepdims=True)
    msa_w = (attn[:, :, None] * m).sum(0)
    node = (_dot(msa_w, exw1_ref[...]) + _dot(seq_ref[...], exw2_ref[...])
            + exb_ref[0])
    out_ref[...] = _elu(node)


def _edge_kernel(pair_ref, seqsep_ref, neg_ref, neb_ref, eew1_ref, eew2_ref,
                 eeb_ref, out_ref):
    p = _lnorm(pair_ref[...], neg_ref[0], neb_ref[0])
    ti = p.shape[0]
    emb = _dot(p.reshape(ti * L, D_EDGE_IN), eew1_ref[...]).reshape(ti, L, D_EHID)
    emb = emb + seqsep_ref[...][:, :, None] * eew2_ref[0] + eeb_ref[0]
    out_ref[...] = _elu(emb)


def _block_kernel(x0_ref, x0t_ref, e_ref, mask_ref, wq_ref, bq_ref, wk_ref,
                  bk_ref, wv_ref, bv_ref, we_ref, be_ref, ws_ref, bs_ref,
                  lng_ref, lnb_ref, wo_ref, bo_ref, xyzw_ref, xyzb_ref,
                  xyz_ref, xa_ref, xb_ref):
    b = pl.program_id(0)
    j = pl.program_id(1)
    x = jnp.where(b == 0, x0_ref[...],
                  jnp.where(b == 1, xa_ref[...], xb_ref[...]))
    x_t = jnp.where(b == 0, x0t_ref[...],
                    jnp.where(b == 1, xa_ref[pl.ds(j * _TJ, _TJ), :],
                              xb_ref[pl.ds(j * _TJ, _TJ), :]))
    q_t = _dot(x_t, wq_ref[0]) + bq_ref[0, 0]
    k = _dot(x, wk_ref[0]) + bk_ref[0, 0]
    v = _dot(x, wv_ref[0]) + bv_ref[0, 0]
    e = e_ref[...]
    maskf = mask_ref[0]
    ep = (_dot(e.reshape(L * _TJ, D_EHID), we_ref[0])
          + be_ref[0, 0]).reshape(L, _TJ, HD)
    ci = jax.lax.broadcasted_iota(jnp.int32, (HD, H), 0)
    hi = jax.lax.broadcasted_iota(jnp.int32, (HD, H), 1)
    sel = (ci // D_HID == hi).astype(jnp.float32)
    kje = k[:, None, :] + ep
    t = kje * q_t[None, :, :]
    al = _dot(t.reshape(L * _TJ, HD), sel).reshape(L, _TJ, H) * (1.0 / 8.0)
    al = jnp.where(maskf[:, :, None] > 0.5, al, -jnp.inf)
    amax = al.max(axis=0, keepdims=True)
    ex = jnp.exp(al - amax)
    den = ex.sum(axis=0, keepdims=True)
    w = ex / (den + 1e-16)
    wb = _dot(w.reshape(L * _TJ, H), sel.T).reshape(L, _TJ, HD)
    out = ((v[:, None, :] + ep) * wb).sum(0)
    out = out + _dot(x_t, ws_ref[0]) + bs_ref[0, 0]
    out = _lnorm(out, lng_ref[0, 0], lnb_ref[0, 0])
    out = _dot(out, wo_ref[0]) + bo_ref[0, 0]
    xn = _elu(out + x_t)

    @pl.when(b == 0)
    def _():
        xa_ref[pl.ds(j * _TJ, _TJ), :] = xn

    @pl.when(b == 1)
    def _():
        xb_ref[pl.ds(j * _TJ, _TJ), :] = xn

    xyz_ref[...] = _dot(xn, xyzw_ref[...]) + xyzb_ref[0]


def _full(shape):
    return pl.BlockSpec(shape, lambda j: tuple(0 for _ in shape))


def kernel(seq1hot, idx, msa, pair, params):
    p = params
    seq = seq1hot[0]
    m0 = msa[0]
    pair0 = pair[0]
    idx0 = idx[0].astype(jnp.int32)
    sep = (idx0[None, :] - idx0[:, None]).astype(jnp.float32)
    seqsep = jnp.sign(sep) * jnp.clip(jnp.log(jnp.abs(sep) + 1.0), 0.0, 5.5)
    ntj = L // _TJ
    idx_i = idx0.reshape(1, L, 1)
    idx_j = idx0.reshape(ntj, 1, _TJ)
    maskt = (jnp.abs(idx_j - idx_i) > 0).astype(jnp.float32)

    r1 = lambda a: a.reshape(1, -1)

    tl = 64
    x = pl.pallas_call(
        _node_kernel,
        grid=(L // tl,),
        in_specs=[
            pl.BlockSpec((N, tl, D_NODE), lambda l: (0, l, 0)),
            pl.BlockSpec((tl, 21), lambda l: (l, 0)),
            _full((1, D_NODE)), _full((1, D_NODE)),
            _full((D_NODE, D_NODE)), _full((1, D_NODE)),
            _full((D_NODE, D_NODE)), _full((1, D_NODE)),
            _full((D_NODE, D_HID)), _full((21, D_HID)), _full((1, D_HID)),
        ],
        out_specs=pl.BlockSpec((tl, D_HID), lambda l: (l, 0)),
        out_shape=jax.ShapeDtypeStruct((L, D_HID), jnp.float32),
    )(m0, seq, r1(p['nn_g']), r1(p['nn_b']), p['sw_Wq'], r1(p['sw_bq']),
      p['sw_Wk'], r1(p['sw_bk']), p['ex_W'][:D_NODE], p['ex_W'][D_NODE:],
      r1(p['ex_b']))

    nti = L // _TI
    e_attr = pl.pallas_call(
        _edge_kernel,
        grid=(nti,),
        in_specs=[
            pl.BlockSpec((_TI, L, D_EDGE_IN), lambda i: (i, 0, 0)),
            pl.BlockSpec((_TI, L), lambda i: (i, 0)),
            _full((1, D_EDGE_IN)), _full((1, D_EDGE_IN)),
            _full((D_EDGE_IN, D_EHID)), _full((1, D_EHID)), _full((1, D_EHID)),
        ],
        out_specs=pl.BlockSpec((_TI, L, D_EHID), lambda i: (i, 0, 0)),
        out_shape=jax.ShapeDtypeStruct((L, L, D_EHID), jnp.float32),
    )(pair0, seqsep, r1(p['ne_g']), r1(p['ne_b']),
      p['ee_W'][:D_EDGE_IN], p['ee_W'][D_EDGE_IN:], r1(p['ee_b']))

    blocks = p['blocks']
    stk = lambda name: jnp.stack([blk[name] for blk in blocks])
    stkb = lambda name: jnp.stack([blk[name].reshape(1, -1) for blk in blocks])

    wmat = lambda shape: pl.BlockSpec((1,) + shape, lambda b, j: (b, 0, 0))
    xyz = pl.pallas_call(
        _block_kernel,
        grid=(len(blocks), ntj),
        in_specs=[
            pl.BlockSpec((L, D_HID), lambda b, j: (0, 0)),
            pl.BlockSpec((_TJ, D_HID), lambda b, j: (j, 0)),
            pl.BlockSpec((L, _TJ, D_EHID), lambda b, j: (0, j, 0)),
            pl.BlockSpec((1, L, _TJ), lambda b, j: (j, 0, 0)),
            wmat((D_HID, HD)), wmat((1, HD)),
            wmat((D_HID, HD)), wmat((1, HD)),
            wmat((D_HID, HD)), wmat((1, HD)),
            wmat((D_EHID, HD)), wmat((1, HD)),
            wmat((D_HID, HD)), wmat((1, HD)),
            wmat((1, HD)), wmat((1, HD)),
            wmat((HD, D_HID)), wmat((1, D_HID)),
            pl.BlockSpec((D_HID, 9), lambda b, j: (0, 0)),
            pl.BlockSpec((1, 9), lambda b, j: (0, 0)),
        ],
        out_specs=pl.BlockSpec((_TJ, 9), lambda b, j: (j, 0)),
        out_shape=jax.ShapeDtypeStruct((L, 9), jnp.float32),
        scratch_shapes=[
            pltpu.VMEM((L, D_HID), jnp.float32),
            pltpu.VMEM((L, D_HID), jnp.float32),
        ],
    )(x, x, e_attr, maskt, stk('Wq'), stkb('bq'), stk('Wk'), stkb('bk'),
      stk('Wv'), stkb('bv'), stk('We'), stkb('be'),
      stk('Ws'), stkb('bs'), stkb('ln_g'), stkb('ln_b'),
      stk('Wo'), stkb('bo'), p['xyz_W'], r1(p['xyz_b']))

    return xyz.reshape(B, L, 3, 3)


# speedup vs baseline: 1.0800x; 1.0800x over previous
import jax
import jax.numpy as jnp
import numpy as np
from jax.experimental import pallas as pl
from jax.experimental.pallas import tpu as pltpu

B, N, L = 1, 128, 256
D_NODE, D_EDGE_IN, D_HID, D_EHID, H = 64, 128, 64, 64, 4
HD = H * D_HID

_PREC = jax.lax.Precision.DEFAULT
_TI = 64
_TJ = 64


def _elu(x):
    return jnp.where(x > 0, x, jnp.exp(x) - 1.0)


def _lnorm(x, g, b, eps=1e-5):
    m = x.mean(-1, keepdims=True)
    v = ((x - m) ** 2).mean(-1, keepdims=True)
    return (x - m) * jax.lax.rsqrt(v + eps) * g + b


def _dot(a, b):
    return jax.lax.dot_general(a, b, (((a.ndim - 1,), (0,)), ((), ())),
                               precision=_PREC,
                               preferred_element_type=jnp.float32)


def _dg(a, b, ca, cb):
    return jax.lax.dot_general(a, b, (((ca,), (cb,)), ((), ())),
                               precision=_PREC,
                               preferred_element_type=jnp.float32)


def _node_kernel(msa_ref, seq_ref, nng_ref, nnb_ref, swq_ref, swbq_ref,
                 swk_ref, swbk_ref, exw1_ref, exw2_ref, exb_ref, out_ref):
    tl = msa_ref.shape[1]
    m = _lnorm(msa_ref[...], nng_ref[0], nnb_ref[0])
    tar = m[0]
    q = _dot(tar, swq_ref[...]) + swbq_ref[0]
    k = (_dot(m.reshape(N * tl, D_NODE), swk_ref[...])
         + swbk_ref[0]).reshape(N, tl, D_NODE)
    logits = (q[None, :, :] * k).sum(-1) * (1.0 / 8.0)
    amax = logits.max(axis=0, keepdims=True)
    ex = jnp.exp(logits - amax)
    attn = ex / ex.sum(axis=0, keepdims=True)
    msa_w = (attn[:, :, None] * m).sum(0)
    node = (_dot(msa_w, exw1_ref[...]) + _dot(seq_ref[...], exw2_ref[...])
            + exb_ref[0])
    out_ref[...] = _elu(node)


def _edge_kernel(pair_ref, seqsep_ref, neg_ref, neb_ref, eew1_ref, eew2_ref,
                 eeb_ref, out_ref):
    p = _lnorm(pair_ref[...], neg_ref[0], neb_ref[0])
    ti = p.shape[0]
    emb = _dot(p.reshape(ti * L, D_EDGE_IN), eew1_ref[...]).reshape(ti, L, D_EHID)
    emb = emb + seqsep_ref[...][:, :, None] * eew2_ref[0] + eeb_ref[0]
    out_ref[...] = _elu(emb)


def _block_kernel(x0_ref, x0t_ref, e_ref, mask_ref, wq_ref, bq_ref, wk_ref,
                  bk_ref, wv_ref, bv_ref, we_ref, be_ref, ws_ref, bs_ref,
                  lng_ref, lnb_ref, wo_ref, bo_ref, xyzw_ref, xyzb_ref,
                  xyz_ref, xa_ref, xb_ref):
    b = pl.program_id(0)
    j = pl.program_id(1)
    x = jnp.where(b == 0, x0_ref[...],
                  jnp.where(b == 1, xa_ref[...], xb_ref[...]))
    x_t = jnp.where(b == 0, x0t_ref[...],
                    jnp.where(b == 1, xa_ref[pl.ds(j * _TJ, _TJ), :],
                              xb_ref[pl.ds(j * _TJ, _TJ), :]))
    q_t = _dot(x_t, wq_ref[0]) + bq_ref[0, 0]
    k = _dot(x, wk_ref[0]) + bk_ref[0, 0]
    v = _dot(x, wv_ref[0]) + bv_ref[0, 0]
    e = e_ref[...]
    maskf = mask_ref[0]
    ep = (_dot(e.reshape(L * _TJ, D_EHID), we_ref[0])
          + be_ref[0, 0]).reshape(L, _TJ, HD)
    ci = jax.lax.broadcasted_iota(jnp.int32, (HD, H), 0)
    hi = jax.lax.broadcasted_iota(jnp.int32, (HD, H), 1)
    sel = (ci // D_HID == hi).astype(jnp.float32)
    kje = k[:, None, :] + ep
    t = kje * q_t[None, :, :]
    al = _dot(t.reshape(L * _TJ, HD), sel).reshape(L, _TJ, H) * (1.0 / 8.0)
    al = jnp.where(maskf[:, :, None] > 0.5, al, -jnp.inf)
    amax = al.max(axis=0, keepdims=True)
    ex = jnp.exp(al - amax)
    den = ex.sum(axis=0, keepdims=True)
    w = ex / (den + 1e-16)
    wb = _dot(w.reshape(L * _TJ, H), sel.T).reshape(L, _TJ, HD)
    out = ((v[:, None, :] + ep) * wb).sum(0)
    out = out + _dot(x_t, ws_ref[0]) + bs_ref[0, 0]
    out = _lnorm(out, lng_ref[0, 0], lnb_ref[0, 0])
    out = _dot(out, wo_ref[0]) + bo_ref[0, 0]
    xn = _elu(out + x_t)

    @pl.when(b == 0)
    def _():
        xa_ref[pl.ds(j * _TJ, _TJ), :] = xn

    @pl.when(b == 1)
    def _():
        xb_ref[pl.ds(j * _TJ, _TJ), :] = xn

    xyz_ref[...] = _dot(xn, xyzw_ref[...]) + xyzb_ref[0]


def _full(shape):
    return pl.BlockSpec(shape, lambda j: tuple(0 for _ in shape))


def kernel(seq1hot, idx, msa, pair, params):
    p = params
    seq = seq1hot[0]
    m0 = msa[0]
    pair0 = pair[0]
    idx0 = idx[0].astype(jnp.int32)
    sep = (idx0[None, :] - idx0[:, None]).astype(jnp.float32)
    seqsep = jnp.sign(sep) * jnp.clip(jnp.log(jnp.abs(sep) + 1.0), 0.0, 5.5)
    ntj = L // _TJ
    idx_i = idx0.reshape(1, L, 1)
    idx_j = idx0.reshape(ntj, 1, _TJ)
    maskt = (jnp.abs(idx_j - idx_i) > 0).astype(jnp.float32)

    r1 = lambda a: a.reshape(1, -1)

    tl = 64
    x = pl.pallas_call(
        _node_kernel,
        grid=(L // tl,),
        in_specs=[
            pl.BlockSpec((N, tl, D_NODE), lambda l: (0, l, 0)),
            pl.BlockSpec((tl, 21), lambda l: (l, 0)),
            _full((1, D_NODE)), _full((1, D_NODE)),
            _full((D_NODE, D_NODE)), _full((1, D_NODE)),
            _full((D_NODE, D_NODE)), _full((1, D_NODE)),
            _full((D_NODE, D_HID)), _full((21, D_HID)), _full((1, D_HID)),
        ],
        out_specs=pl.BlockSpec((tl, D_HID), lambda l: (l, 0)),
        out_shape=jax.ShapeDtypeStruct((L, D_HID), jnp.float32),
    )(m0, seq, r1(p['nn_g']), r1(p['nn_b']), p['sw_Wq'], r1(p['sw_bq']),
      p['sw_Wk'], r1(p['sw_bk']), p['ex_W'][:D_NODE], p['ex_W'][D_NODE:],
      r1(p['ex_b']))

    nti = L // _TI
    e_attr = pl.pallas_call(
        _edge_kernel,
        grid=(nti,),
        in_specs=[
            pl.BlockSpec((_TI, L, D_EDGE_IN), lambda i: (i, 0, 0)),
            pl.BlockSpec((_TI, L), lambda i: (i, 0)),
            _full((1, D_EDGE_IN)), _full((1, D_EDGE_IN)),
            _full((D_EDGE_IN, D_EHID)), _full((1, D_EHID)), _full((1, D_EHID)),
        ],
        out_specs=pl.BlockSpec((_TI, L, D_EHID), lambda i: (i, 0, 0)),
        out_shape=jax.ShapeDtypeStruct((L, L, D_EHID), jnp.float32),
    )(pair0, seqsep, r1(p['ne_g']), r1(p['ne_b']),
      p['ee_W'][:D_EDGE_IN], p['ee_W'][D_EDGE_IN:], r1(p['ee_b']))

    blocks = p['blocks']
    stk = lambda name: jnp.stack([blk[name] for blk in blocks])
    stkb = lambda name: jnp.stack([blk[name].reshape(1, -1) for blk in blocks])

    wmat = lambda shape: pl.BlockSpec((1,) + shape, lambda b, j: (b, 0, 0))
    xyz = pl.pallas_call(
        _block_kernel,
        grid=(len(blocks), ntj),
        in_specs=[
            pl.BlockSpec((L, D_HID), lambda b, j: (0, 0)),
            pl.BlockSpec((_TJ, D_HID), lambda b, j: (j, 0)),
            pl.BlockSpec((L, _TJ, D_EHID), lambda b, j: (0, j, 0)),
            pl.BlockSpec((1, L, _TJ), lambda b, j: (j, 0, 0)),
            wmat((D_HID, HD)), wmat((1, HD)),
            wmat((D_HID, HD)), wmat((1, HD)),
            wmat((D_HID, HD)), wmat((1, HD)),
            wmat((D_EHID, HD)), wmat((1, HD)),
            wmat((D_HID, HD)), wmat((1, HD)),
            wmat((1, HD)), wmat((1, HD)),
            wmat((HD, D_HID)), wmat((1, D_HID)),
            pl.BlockSpec((D_HID, 9), lambda b, j: (0, 0)),
            pl.BlockSpec((1, 9), lambda b, j: (0, 0)),
        ],
        out_specs=pl.BlockSpec((_TJ, 9), lambda b, j: (j, 0)),
        out_shape=jax.ShapeDtypeStruct((L, 9), jnp.float32),
        scratch_shapes=[
            pltpu.VMEM((L, D_HID), jnp.float32),
            pltpu.VMEM((L, D_HID), jnp.float32),
        ],
    )(x, x, e_attr, maskt, stk('Wq'), stkb('bq'), stk('Wk'), stkb('bk'),
      stk('Wv'), stkb('bv'), stk('We'), stkb('be'),
      stk('Ws'), stkb('bs'), stkb('ln_g'), stkb('ln_b'),
      stk('Wo'), stkb('bo'), p['xyz_W'], r1(p['xyz_b']))

    return xyz.reshape(B, L, 3, 3)
